# baseline (speedup 1.0000x reference)
import jax
import jax.numpy as jnp
from jax import lax
from jax.experimental import pallas as pl
from jax.experimental.pallas import tpu as pltpu
from jax.experimental.pallas import tpu_sc as plsc

M = 2048
D = 128
N = 8192
TN = 4096
NB = N // TN

NC = 2
NS = 16
NW = NC * NS
TPW = N // NW
RPS = M // NS


def _score_body(q_ref, v_ref, k_ref, uv_ref, g_ref, c_ref, cmacc):
    b = pl.program_id(0)

    q = q_ref[...]
    ss = jnp.sum(q * q, axis=1, keepdims=True)
    qn = q / jnp.maximum(jnp.sqrt(ss), 1e-12)
    score = lax.dot_general(
        k_ref[...], qn, (((1,), (1,)), ((), ())),
        preferred_element_type=jnp.float32)
    smax = jnp.max(score, axis=0, keepdims=True)
    iota0 = lax.broadcasted_iota(jnp.int32, (M, TN), 0)
    g = jnp.min(jnp.where(score == smax, iota0, M),
                axis=0, keepdims=True)
    e = jnp.exp(smax)
    ecol = jnp.transpose(e, (1, 0))
    uv_ref[...] = v_ref[...] * ecol
    g_ref[...] = g[0, :]
    cpart = jnp.max(score, axis=1, keepdims=True)

    @pl.when(b == 0)
    def _():
        cmacc[...] = cpart

    @pl.when(b > 0)
    def _():
        cmacc[...] = jnp.maximum(cmacc[...], cpart)

    @pl.when(b == NB - 1)
    def _():
        c_ref[...] = jnp.maximum(cmacc[...], cpart)[:, 0]


def _sc_scatter_body(uv_hbm, g_hbm, zero_hbm, out_hbm, shacc, idx_v, rows_v):
    cid = lax.axis_index("c")
    sid = lax.axis_index("s")
    wid = cid * NS + sid
    base = wid * TPW

    pltpu.sync_copy(zero_hbm.at[pl.ds(sid * RPS, RPS), :],
                    shacc.at[pl.ds(sid * RPS, RPS), :])
    plsc.subcore_barrier()

    pltpu.sync_copy(g_hbm.at[pl.ds(wid * 2, 2), :], idx_v)
    for j in range(TPW // 128):
        pltpu.sync_copy(uv_hbm.at[pl.ds(base + j * 128, 128), :], rows_v)
        pltpu.sync_copy(rows_v, shacc.at[idx_v.at[j]], add=True)

    plsc.subcore_barrier()
    pltpu.sync_copy(shacc.at[pl.ds(sid * RPS, RPS), :],
                    out_hbm.at[cid, pl.ds(sid * RPS, RPS), :])


def _finish_body(a_ref, c_ref, k_ref, out_ref):
    a = a_ref[0] + a_ref[1]
    c = c_ref[...][:, None]
    mem = a * jnp.exp(-c) + k_ref[...]
    nn = jnp.sqrt(jnp.sum(mem * mem, axis=1, keepdims=True))
    out_ref[...] = mem / jnp.maximum(nn, 1e-12)


def kernel(keys, query, value):
    qf = jnp.transpose(query, (0, 2, 3, 1)).reshape(N, D)
    vf = jnp.transpose(value, (0, 2, 3, 1)).reshape(N, D)

    uv, g, cmax = pl.pallas_call(
        _score_body,
        grid=(NB,),
        in_specs=[
            pl.BlockSpec((TN, D), lambda b: (b, 0)),
            pl.BlockSpec((TN, D), lambda b: (b, 0)),
            pl.BlockSpec((M, D), lambda b: (0, 0)),
        ],
        out_specs=[
            pl.BlockSpec((TN, D), lambda b: (b, 0)),
            pl.BlockSpec((TN,), lambda b: (b,)),
            pl.BlockSpec((M,), lambda b: (0,)),
        ],
        out_shape=[
            jax.ShapeDtypeStruct((N, D), jnp.float32),
            jax.ShapeDtypeStruct((N,), jnp.int32),
            jax.ShapeDtypeStruct((M,), jnp.float32),
        ],
        scratch_shapes=[
            pltpu.VMEM((M, 1), jnp.float32),
        ],
    )(qf, vf, keys)

    g2 = g.reshape(N // 128, 128)
    zeros = jnp.zeros((M, D), jnp.float32)

    mesh = plsc.VectorSubcoreMesh(core_axis_name="c", subcore_axis_name="s")
    accs = pl.kernel(
        _sc_scatter_body,
        out_type=jax.ShapeDtypeStruct((NC, M, D), jnp.float32),
        mesh=mesh,
        scratch_types=[
            pltpu.VMEM_SHARED((M, D), jnp.float32),
            pltpu.VMEM((2, 128), jnp.int32),
            pltpu.VMEM((128, D), jnp.float32),
        ],
    )(uv, g2, zeros)

    out = pl.pallas_call(
        _finish_body,
        grid=(1,),
        in_specs=[
            pl.BlockSpec((NC, M, D), lambda b: (0, 0, 0)),
            pl.BlockSpec((M,), lambda b: (0,)),
            pl.BlockSpec((M, D), lambda b: (0, 0)),
        ],
        out_specs=pl.BlockSpec((M, D), lambda b: (0, 0)),
        out_shape=jax.ShapeDtypeStruct((M, D), jnp.float32),
    )(accs, cmax, keys)
    return out


# speedup vs baseline: 1.0195x; 1.0195x over previous
import jax
import jax.numpy as jnp
from jax import lax
from jax.experimental import pallas as pl
from jax.experimental.pallas import tpu as pltpu

M = 2048
D = 128
N = 8192
TN = 4096
NB = N // TN


def _body(q_ref, v_ref, k_ref, out_ref, cmacc, acc):
    b = pl.program_id(0)

    q = q_ref[...]
    ss = jnp.sum(q * q, axis=1, keepdims=True)
    qn = q / jnp.maximum(jnp.sqrt(ss), 1e-12)
    score = lax.dot_general(
        k_ref[...], qn, (((1,), (1,)), ((), ())),
        preferred_element_type=jnp.float32)
    smax = jnp.max(score, axis=0, keepdims=True)
    iota0 = lax.broadcasted_iota(jnp.int32, (M, TN), 0)
    g = jnp.min(jnp.where(score == smax, iota0, M),
                axis=0, keepdims=True)
    e = jnp.exp(smax)
    woh = jnp.where(iota0 == g, e, 0.0)
    contrib = lax.dot_general(
        woh, v_ref[...], (((1,), (0,)), ((), ())),
        preferred_element_type=jnp.float32)

    cpart = jnp.max(score.reshape(M, TN // 128, 128), axis=1)

    @pl.when(b == 0)
    def _():
        cmacc[...] = cpart
        acc[...] = contrib

    @pl.when(b > 0)
    def _():
        cmacc[...] = jnp.maximum(cmacc[...], cpart)
        acc[...] += contrib

    @pl.when(b == NB - 1)
    def _():
        cm = jnp.maximum(cmacc[...], cpart)
        c = jnp.max(cm, axis=1, keepdims=True)
        mem = acc[...] * jnp.exp(-c) + k_ref[...]
        nn = jnp.sqrt(jnp.sum(mem * mem, axis=1, keepdims=True))
        out_ref[...] = mem / jnp.maximum(nn, 1e-12)


def kernel(keys, query, value):
    qf = jnp.transpose(query, (0, 2, 3, 1)).reshape(N, D)
    vf = jnp.transpose(value, (0, 2, 3, 1)).reshape(N, D)

    out = pl.pallas_call(
        _body,
        grid=(NB,),
        in_specs=[
            pl.BlockSpec((TN, D), lambda b: (b, 0)),
            pl.BlockSpec((TN, D), lambda b: (b, 0)),
            pl.BlockSpec((M, D), lambda b: (0, 0)),
        ],
        out_specs=pl.BlockSpec((M, D), lambda b: (0, 0)),
        out_shape=jax.ShapeDtypeStruct((M, D), jnp.float32),
        scratch_shapes=[
            pltpu.VMEM((M, 128), jnp.float32),
            pltpu.VMEM((M, D), jnp.float32),
        ],
    )(qf, vf, keys)
    return out


# speedup vs baseline: 1.6322x; 1.6011x over previous
import jax
import jax.numpy as jnp
from jax import lax
from jax.experimental import pallas as pl
from jax.experimental.pallas import tpu as pltpu

M = 2048
D = 128
N = 8192
TN = 4096
NB = N // TN


def _body(q_ref, v_ref, k_ref, out_ref, cmacc, acc):
    b = pl.program_id(0)

    q = q_ref[...]
    ss = jnp.sum(q * q, axis=1, keepdims=True)
    qn = q / jnp.maximum(jnp.sqrt(ss), 1e-12)
    score = lax.dot_general(
        k_ref[...], qn, (((1,), (1,)), ((), ())),
        preferred_element_type=jnp.float32)
    smax = jnp.max(score, axis=0, keepdims=True)
    iota0 = lax.broadcasted_iota(jnp.int32, (M, TN), 0)
    g = jnp.min(jnp.where(score == smax, iota0, M),
                axis=0, keepdims=True)
    e = jnp.exp(smax)
    woh = jnp.where(iota0 == g, e, 0.0)
    contrib = lax.dot_general(
        woh, v_ref[...], (((1,), (0,)), ((), ())),
        preferred_element_type=jnp.float32)

    cpart = jnp.max(score, axis=1, keepdims=True)

    @pl.when(b == 0)
    def _():
        cmacc[...] = cpart
        acc[...] = contrib

    @pl.when(b > 0)
    def _():
        cmacc[...] = jnp.maximum(cmacc[...], cpart)
        acc[...] += contrib

    @pl.when(b == NB - 1)
    def _():
        c = jnp.maximum(cmacc[...], cpart)
        mem = acc[...] * jnp.exp(-c) + k_ref[...]
        nn = jnp.sqrt(jnp.sum(mem * mem, axis=1, keepdims=True))
        out_ref[...] = mem / jnp.maximum(nn, 1e-12)


def kernel(keys, query, value):
    qf = jnp.transpose(query, (0, 2, 3, 1)).reshape(N, D)
    vf = jnp.transpose(value, (0, 2, 3, 1)).reshape(N, D)

    out = pl.pallas_call(
        _body,
        grid=(NB,),
        in_specs=[
            pl.BlockSpec((TN, D), lambda b: (b, 0)),
            pl.BlockSpec((TN, D), lambda b: (b, 0)),
            pl.BlockSpec((M, D), lambda b: (0, 0)),
        ],
        out_specs=pl.BlockSpec((M, D), lambda b: (0, 0)),
        out_shape=jax.ShapeDtypeStruct((M, D), jnp.float32),
        scratch_shapes=[
            pltpu.VMEM((M, 1), jnp.float32),
            pltpu.VMEM((M, D), jnp.float32),
        ],
    )(qf, vf, keys)
    return out
